# baseline (speedup 1.0000x reference)
import functools

import jax
import jax.numpy as jnp
from jax import lax
from jax.experimental import pallas as pl
from jax.experimental.pallas import tpu as pltpu
from jax.experimental.pallas import tpu_sc as plsc

DIM = 64
NEG = 10
L = 16
NK = DIM // L
C = 128


def _sc_scores(x, targets, neg_flat, emb_table, out_weight):
    B = x.shape[0]
    info = plsc.get_sparse_core_info()
    NC, NS = info.num_cores, info.num_subcores
    NW = NC * NS
    per_w = B // NW
    n_chunks = per_w // C

    mesh = plsc.VectorSubcoreMesh(core_axis_name="c", subcore_axis_name="s")

    @functools.partial(
        pl.kernel,
        mesh=mesh,
        compiler_params=pltpu.CompilerParams(
            needs_layout_passes=False, use_tc_tiling_on_sc=False
        ),
        out_type=(
            jax.ShapeDtypeStruct((B,), jnp.float32),
            jax.ShapeDtypeStruct((B * NEG,), jnp.float32),
        ),
        scratch_types=[
            pltpu.VMEM((C,), jnp.int32),
            pltpu.VMEM((C,), jnp.int32),
            pltpu.VMEM((NEG, C), jnp.int32),
            pltpu.VMEM((C, DIM), jnp.float32),
            pltpu.VMEM((C, DIM), jnp.float32),
            pltpu.VMEM((C * NEG, DIM), jnp.float32),
            pltpu.VMEM((C,), jnp.float32),
            pltpu.VMEM((C * NEG,), jnp.float32),
            pltpu.SemaphoreType.DMA,
        ],
    )
    def k(x_h, t_h, n_h, emb_h, w_h, pos_h, negs_h,
          idx_e, idx_p, idx_n, emb_v, pos_v, neg_v, pos_o, neg_o, sem):
        wid = lax.axis_index("s") * NC + lax.axis_index("c")
        base = wid * per_w

        def chunk_body(ci, _):
            cbase = base + ci * C
            pltpu.sync_copy(x_h.at[pl.ds(cbase, C)], idx_e)
            pltpu.sync_copy(t_h.at[pl.ds(cbase, C)], idx_p)
            for j in range(NEG):
                pltpu.sync_copy(n_h.at[pl.ds(cbase * NEG + j * C, C)], idx_n.at[j])
            copies = [
                pltpu.async_copy(emb_h.at[idx_e], emb_v, sem),
                pltpu.async_copy(w_h.at[idx_p], pos_v, sem),
            ]
            for j in range(NEG):
                copies.append(
                    pltpu.async_copy(
                        w_h.at[idx_n.at[j]], neg_v.at[pl.ds(j * C, C)], sem
                    )
                )
            for cp in copies:
                cp.wait()

            def body(g, _):
                rows = g * L + lax.iota(jnp.int32, L)
                nrows = [rows * NEG + j for j in range(NEG)]
                pos_acc = jnp.zeros((L,), jnp.float32)
                neg_accs = [jnp.zeros((L,), jnp.float32) for _ in range(NEG)]
                for d in range(DIM):
                    cold = jnp.full((L,), d, jnp.int32)
                    ev = plsc.load_gather(emb_v, [rows, cold])
                    pv = plsc.load_gather(pos_v, [rows, cold])
                    pos_acc = pos_acc + ev * pv
                    for j in range(NEG):
                        nv = plsc.load_gather(neg_v, [nrows[j], cold])
                        neg_accs[j] = neg_accs[j] + ev * nv
                pos_o[pl.ds(g * L, L)] = pos_acc
                for j in range(NEG):
                    neg_o[pl.ds(j * C + g * L, L)] = neg_accs[j]
                return 0

            lax.fori_loop(0, C // L, body, 0)
            pltpu.sync_copy(pos_o, pos_h.at[pl.ds(cbase, C)])
            pltpu.sync_copy(neg_o, negs_h.at[pl.ds(cbase * NEG, C * NEG)])
            return 0

        lax.fori_loop(0, n_chunks, chunk_body, 0)

    return k(x, targets, neg_flat, emb_table, out_weight)


def _tc_loss(pos, neg, B):
    def body(pos_ref, neg_ref, out_ref):
        p = pos_ref[...]
        n = neg_ref[...]
        sp = jnp.maximum(-p, 0.0) + jnp.log1p(jnp.exp(-jnp.abs(p)))
        sn = jnp.maximum(n, 0.0) + jnp.log1p(jnp.exp(-jnp.abs(n)))
        out_ref[...] = ((jnp.sum(sp) + jnp.sum(sn)) * (1.0 / B)).reshape(1, 1)

    res = pl.pallas_call(
        body,
        out_shape=jax.ShapeDtypeStruct((1, 1), jnp.float32),
    )(pos.reshape(B // 128, 128), neg.reshape(B * NEG // 128, 128))
    return res[0, 0]


def kernel(x, targets, negatives, emb_table, out_weight):
    B = x.shape[0]
    x = x.astype(jnp.int32)
    targets = targets.astype(jnp.int32)
    neg_flat = negatives.astype(jnp.int32).reshape(-1)
    pos_s, neg_s = _sc_scores(x, targets, neg_flat, emb_table, out_weight)
    return _tc_loss(pos_s, neg_s, B)
